# Initial kernel scaffold; baseline (speedup 1.0000x reference)
import jax
import jax.numpy as jnp
from jax.experimental import pallas as pl


def kernel(x, edge_index, edge_attr, params):
    raise NotImplementedError("write your pallas kernel here")


# speedup vs baseline: 1.0721x; 1.0721x over previous
import functools

import jax
import jax.numpy as jnp
from jax import lax
from jax.experimental import pallas as pl
from jax.experimental.pallas import tpu as pltpu

D = 256
BN = 2048
BE = 2048


def _f32dot(a, b):
    return jnp.dot(a, b, preferred_element_type=jnp.float32)


def _layer_norm(r, g, b):
    m = jnp.mean(r, axis=-1, keepdims=True)
    d = r - m
    v = jnp.mean(d * d, axis=-1, keepdims=True)
    return d * lax.rsqrt(v + 1e-5) * g + b


def _proj_body(ne_ref, w1s_ref, w1d_ref, g_ref, w1g_ref, b1_ref, v1g_ref,
               bn1_ref, a_ref, b_ref, crow_ref, cnrow_ref):
    ne = ne_ref[...]
    a_ref[...] = _f32dot(ne, w1s_ref[...])
    b_ref[...] = _f32dot(ne, w1d_ref[...])

    @pl.when(pl.program_id(0) == 0)
    def _():
        gl = g_ref[...]
        crow_ref[...] = _f32dot(gl, w1g_ref[...]) + b1_ref[...]
        cnrow_ref[...] = _f32dot(gl, v1g_ref[...]) + bn1_ref[...]


def _node_proj(node_emb, g_emb, w1s, w1d, w1g, b1, v1g, bn1):
    npad = node_emb.shape[0]
    nblk = npad // BN
    row = pl.BlockSpec((1, D), lambda i: (0, 0))
    mat = pl.BlockSpec((D, D), lambda i: (0, 0))
    blk = pl.BlockSpec((BN, D), lambda i: (i, 0))
    return pl.pallas_call(
        _proj_body,
        grid=(nblk,),
        in_specs=[blk, mat, mat, row, mat, row, mat, row],
        out_specs=[blk, blk, row, row],
        out_shape=[
            jax.ShapeDtypeStruct((npad, D), jnp.float32),
            jax.ShapeDtypeStruct((npad, D), jnp.float32),
            jax.ShapeDtypeStruct((1, D), jnp.float32),
            jax.ShapeDtypeStruct((1, D), jnp.float32),
        ],
    )(node_emb, w1s, w1d, g_emb, w1g, b1, v1g, bn1)


def _edge_body(ga_ref, gb_ref, ee_ref, w1e_ref, crow_ref, w2_ref, b2_ref,
               w3_ref, b3_ref, lg_ref, lb_ref, out_ref):
    ee = ee_ref[...]
    h1 = jnp.maximum(
        ga_ref[...] + gb_ref[...] + _f32dot(ee, w1e_ref[...]) + crow_ref[...],
        0.0)
    h2 = jnp.maximum(_f32dot(h1, w2_ref[...]) + b2_ref[...], 0.0)
    u = _f32dot(h2, w3_ref[...]) + b3_ref[...]
    out_ref[...] = _layer_norm(ee + u, lg_ref[...], lb_ref[...])


def _edge_update(ga, gb, ee, w1e, crow, w2, b2, w3, b3, lg, lb):
    epad = ee.shape[0]
    nblk = epad // BE
    row = pl.BlockSpec((1, D), lambda i: (0, 0))
    mat = pl.BlockSpec((D, D), lambda i: (0, 0))
    blk = pl.BlockSpec((BE, D), lambda i: (i, 0))
    return pl.pallas_call(
        _edge_body,
        grid=(nblk,),
        in_specs=[blk, blk, blk, mat, row, mat, row, mat, row, row, row],
        out_specs=blk,
        out_shape=jax.ShapeDtypeStruct((epad, D), jnp.float32),
    )(ga, gb, ee, w1e, crow, w2, b2, w3, b3, lg, lb)


def _node_body(nvalid_ref, ne_ref, s_ref, cnt_ref, wm_ref, bm_ref, v1n_ref,
               v1m_ref, cnrow_ref, v2_ref, b2_ref, v3_ref, b3_ref, lg_ref,
               lb_ref, out_ref, nsum_ref, ssum_ref):
    ne = ne_ref[...]
    s = s_ref[...]
    nm = _f32dot(s, wm_ref[...]) + cnt_ref[...][:, 0:1] * bm_ref[...]
    h1 = jnp.maximum(
        _f32dot(ne, v1n_ref[...]) + _f32dot(nm, v1m_ref[...]) + cnrow_ref[...],
        0.0)
    h2 = jnp.maximum(_f32dot(h1, v2_ref[...]) + b2_ref[...], 0.0)
    u = _f32dot(h2, v3_ref[...]) + b3_ref[...]
    out = _layer_norm(ne + u, lg_ref[...], lb_ref[...])
    out_ref[...] = out

    i = pl.program_id(0)
    rows = i * BN + lax.broadcasted_iota(jnp.int32, (BN, 1), 0)
    mask = rows < nvalid_ref[0]

    @pl.when(i == 0)
    def _():
        nsum_ref[...] = jnp.zeros_like(nsum_ref)
        ssum_ref[...] = jnp.zeros_like(ssum_ref)

    nsum_ref[...] += jnp.sum(jnp.where(mask, out, 0.0), axis=0, keepdims=True)
    ssum_ref[...] += jnp.sum(jnp.where(mask, s, 0.0), axis=0, keepdims=True)


def _node_update(nvalid, ne, s, cnt, wm, bm, v1n, v1m, cnrow, v2, b2, v3, b3,
                 lg, lb):
    npad = ne.shape[0]
    nblk = npad // BN
    row = pl.BlockSpec((1, D), lambda i: (0, 0))
    mat = pl.BlockSpec((D, D), lambda i: (0, 0))
    blk = pl.BlockSpec((BN, D), lambda i: (i, 0))
    cblk = pl.BlockSpec((BN, 128), lambda i: (i, 0))
    return pl.pallas_call(
        _node_body,
        grid=(nblk,),
        in_specs=[pl.BlockSpec(memory_space=pltpu.SMEM),
                  blk, blk, cblk, mat, row, mat, mat, row, mat, row, mat, row,
                  row, row],
        out_specs=[blk, row, row],
        out_shape=[
            jax.ShapeDtypeStruct((npad, D), jnp.float32),
            jax.ShapeDtypeStruct((1, D), jnp.float32),
            jax.ShapeDtypeStruct((1, D), jnp.float32),
        ],
    )(nvalid, ne, s, cnt, wm, bm, v1n, v1m, cnrow, v2, b2, v3, b3, lg, lb)


def _global_body(nreal_ref, ereal_ref, g_ref, ns_ref, ss_ref, wng_ref,
                 bng_ref, weg_ref, beg_ref, u1g_ref, u1n_ref, u1e_ref, ub1_ref,
                 u2_ref, ub2_ref, u3_ref, ub3_ref, lg_ref, lb_ref, out_ref):
    gl = g_ref[...]
    na = _f32dot(ns_ref[...], wng_ref[...]) + nreal_ref[0] * bng_ref[...]
    ea = _f32dot(ss_ref[...], weg_ref[...]) + ereal_ref[0] * beg_ref[...]
    h1 = jnp.maximum(
        _f32dot(gl, u1g_ref[...]) + _f32dot(na, u1n_ref[...]) +
        _f32dot(ea, u1e_ref[...]) + ub1_ref[...], 0.0)
    h2 = jnp.maximum(_f32dot(h1, u2_ref[...]) + ub2_ref[...], 0.0)
    u = _f32dot(h2, u3_ref[...]) + ub3_ref[...]
    out_ref[...] = _layer_norm(gl + u, lg_ref[...], lb_ref[...])


def _global_update(nreal, ereal, g_emb, nsum, ssum, wng, bng, weg, beg, u1g,
                   u1n, u1e, ub1, u2, ub2, u3, ub3, lg, lb):
    row = pl.BlockSpec((1, D), lambda: (0, 0))
    mat = pl.BlockSpec((D, D), lambda: (0, 0))
    return pl.pallas_call(
        _global_body,
        grid=(),
        in_specs=[pl.BlockSpec(memory_space=pltpu.SMEM),
                  pl.BlockSpec(memory_space=pltpu.SMEM),
                  row, row, row, mat, row, mat, row, mat, mat, mat, row, mat,
                  row, mat, row, row, row],
        out_specs=row,
        out_shape=jax.ShapeDtypeStruct((1, D), jnp.float32),
    )(nreal, ereal, g_emb, nsum, ssum, wng, bng, weg, beg, u1g, u1n, u1e, ub1,
      u2, ub2, u3, ub3, lg, lb)


def _gather2(a, b, src, dst):
    return a[src], b[dst]


def _segment_sum(vals, dst, npad):
    return jax.ops.segment_sum(vals, dst, num_segments=npad)


def kernel(x, edge_index, edge_attr, params):
    n = x.shape[0]
    e = edge_index.shape[1]
    npad = ((n + BN - 1) // BN) * BN
    epad = ((e + BE - 1) // BE) * BE

    src = jnp.pad(edge_index[0], (0, epad - e))
    dst = jnp.pad(edge_index[1], (0, epad - e), constant_values=n)
    eattr = jnp.pad(edge_attr[:, 0], (0, epad - e))
    x0 = jnp.pad(x[:, 0], (0, npad - n))
    x1 = jnp.pad(x[:, 1], (0, npad - n))

    node_emb = jnp.concatenate(
        [params["node_type_emb"][x0], params["node_token_emb"][x1]], axis=1)
    edge_emb = params["edge_type_emb"][eattr]
    gi = params["global_init"]
    g_emb = (jnp.ones((1, 1), jnp.float32) @ gi["w"] + gi["b"])

    counts = _segment_sum(jnp.ones((epad, 128), jnp.float32), dst, npad)

    nreal = jnp.array([n], jnp.float32)
    ereal = jnp.array([e], jnp.float32)
    nvalid = jnp.array([n], jnp.int32)

    for lp in params["layers"]:
        w1 = lp["edge_mlp"][0]["w"]
        w1s, w1d, w1e, w1g = w1[:D], w1[D:2 * D], w1[2 * D:3 * D], w1[3 * D:]
        b1 = lp["edge_mlp"][0]["b"][None, :]
        v1 = lp["node_mlp"][0]["w"]
        v1n, v1m, v1g = v1[:D], v1[D:2 * D], v1[2 * D:]
        bn1 = lp["node_mlp"][0]["b"][None, :]
        u1 = lp["global_mlp"][0]["w"]
        u1g, u1n, u1e = u1[:D], u1[D:2 * D], u1[2 * D:]

        a, b, crow, cnrow = _node_proj(node_emb, g_emb, w1s, w1d, w1g, b1,
                                       v1g, bn1)
        ga, gb = _gather2(a, b, src, dst)
        edge_emb = _edge_update(
            ga, gb, edge_emb, w1e, crow,
            lp["edge_mlp"][1]["w"], lp["edge_mlp"][1]["b"][None, :],
            lp["edge_mlp"][2]["w"], lp["edge_mlp"][2]["b"][None, :],
            lp["ln_edge"]["g"][None, :], lp["ln_edge"]["b"][None, :])
        s = _segment_sum(edge_emb, dst, npad)
        node_emb, nsum, ssum = _node_update(
            nvalid, node_emb, s, counts,
            lp["edge_to_message"]["w"], lp["edge_to_message"]["b"][None, :],
            v1n, v1m, cnrow,
            lp["node_mlp"][1]["w"], lp["node_mlp"][1]["b"][None, :],
            lp["node_mlp"][2]["w"], lp["node_mlp"][2]["b"][None, :],
            lp["ln_node"]["g"][None, :], lp["ln_node"]["b"][None, :])
        g_emb = _global_update(
            nreal, ereal, g_emb, nsum, ssum,
            lp["node_to_global"]["w"], lp["node_to_global"]["b"][None, :],
            lp["edge_to_global"]["w"], lp["edge_to_global"]["b"][None, :],
            u1g, u1n, u1e, lp["global_mlp"][0]["b"][None, :],
            lp["global_mlp"][1]["w"], lp["global_mlp"][1]["b"][None, :],
            lp["global_mlp"][2]["w"], lp["global_mlp"][2]["b"][None, :],
            lp["ln_global"]["g"][None, :], lp["ln_global"]["b"][None, :])

    return node_emb[:n], edge_emb[:e], g_emb


# speedup vs baseline: 1.9494x; 1.8182x over previous
import functools

import jax
import jax.numpy as jnp
from jax import lax
from jax.experimental import pallas as pl
from jax.experimental.pallas import tpu as pltpu
from jax.experimental.pallas import tpu_sc as plsc

D = 256
BN = 2048
BE = 2048
CH = 128
NW = 32
NT = 16


def _sc_mesh():
    return plsc.VectorSubcoreMesh(core_axis_name="c", subcore_axis_name="s")


def _f32dot(a, b):
    return jnp.dot(a, b, preferred_element_type=jnp.float32)


def _layer_norm(r, g, b):
    m = jnp.mean(r, axis=-1, keepdims=True)
    d = r - m
    v = jnp.mean(d * d, axis=-1, keepdims=True)
    return d * lax.rsqrt(v + 1e-5) * g + b


def _proj_body(ne_ref, w1s_ref, w1d_ref, g_ref, w1g_ref, b1_ref, v1g_ref,
               bn1_ref, a_ref, b_ref, crow_ref, cnrow_ref):
    ne = ne_ref[...]
    a_ref[...] = _f32dot(ne, w1s_ref[...])
    b_ref[...] = _f32dot(ne, w1d_ref[...])

    @pl.when(pl.program_id(0) == 0)
    def _():
        gl = g_ref[...]
        crow_ref[...] = _f32dot(gl, w1g_ref[...]) + b1_ref[...]
        cnrow_ref[...] = _f32dot(gl, v1g_ref[...]) + bn1_ref[...]


def _node_proj(node_emb, g_emb, w1s, w1d, w1g, b1, v1g, bn1):
    npad = node_emb.shape[0]
    nblk = npad // BN
    row = pl.BlockSpec((1, D), lambda i: (0, 0))
    mat = pl.BlockSpec((D, D), lambda i: (0, 0))
    blk = pl.BlockSpec((BN, D), lambda i: (i, 0))
    return pl.pallas_call(
        _proj_body,
        grid=(nblk,),
        in_specs=[blk, mat, mat, row, mat, row, mat, row],
        out_specs=[blk, blk, row, row],
        out_shape=[
            jax.ShapeDtypeStruct((npad, D), jnp.float32),
            jax.ShapeDtypeStruct((npad, D), jnp.float32),
            jax.ShapeDtypeStruct((1, D), jnp.float32),
            jax.ShapeDtypeStruct((1, D), jnp.float32),
        ],
    )(node_emb, w1s, w1d, g_emb, w1g, b1, v1g, bn1)


def _edge_body(ga_ref, gb_ref, ee_ref, w1e_ref, crow_ref, w2_ref, b2_ref,
               w3_ref, b3_ref, lg_ref, lb_ref, out_ref):
    ee = ee_ref[...]
    h1 = jnp.maximum(
        ga_ref[...] + gb_ref[...] + _f32dot(ee, w1e_ref[...]) + crow_ref[...],
        0.0)
    h2 = jnp.maximum(_f32dot(h1, w2_ref[...]) + b2_ref[...], 0.0)
    u = _f32dot(h2, w3_ref[...]) + b3_ref[...]
    out_ref[...] = _layer_norm(ee + u, lg_ref[...], lb_ref[...])


def _edge_update(ga, gb, ee, w1e, crow, w2, b2, w3, b3, lg, lb):
    epad = ee.shape[0]
    nblk = epad // BE
    row = pl.BlockSpec((1, D), lambda i: (0, 0))
    mat = pl.BlockSpec((D, D), lambda i: (0, 0))
    blk = pl.BlockSpec((BE, D), lambda i: (i, 0))
    return pl.pallas_call(
        _edge_body,
        grid=(nblk,),
        in_specs=[blk, blk, blk, mat, row, mat, row, mat, row, row, row],
        out_specs=blk,
        out_shape=jax.ShapeDtypeStruct((epad, D), jnp.float32),
    )(ga, gb, ee, w1e, crow, w2, b2, w3, b3, lg, lb)


def _node_body(nvalid_ref, ne_ref, s_ref, cnt_ref, wm_ref, bm_ref, v1n_ref,
               v1m_ref, cnrow_ref, v2_ref, b2_ref, v3_ref, b3_ref, lg_ref,
               lb_ref, out_ref, nsum_ref, ssum_ref):
    ne = ne_ref[...]
    s = s_ref[...]
    nm = _f32dot(s, wm_ref[...]) + cnt_ref[...][:, 0:1] * bm_ref[...]
    h1 = jnp.maximum(
        _f32dot(ne, v1n_ref[...]) + _f32dot(nm, v1m_ref[...]) + cnrow_ref[...],
        0.0)
    h2 = jnp.maximum(_f32dot(h1, v2_ref[...]) + b2_ref[...], 0.0)
    u = _f32dot(h2, v3_ref[...]) + b3_ref[...]
    out = _layer_norm(ne + u, lg_ref[...], lb_ref[...])
    out_ref[...] = out

    i = pl.program_id(0)
    rows = i * BN + lax.broadcasted_iota(jnp.int32, (BN, 1), 0)
    mask = rows < nvalid_ref[0]

    @pl.when(i == 0)
    def _():
        nsum_ref[...] = jnp.zeros_like(nsum_ref)
        ssum_ref[...] = jnp.zeros_like(ssum_ref)

    nsum_ref[...] += jnp.sum(jnp.where(mask, out, 0.0), axis=0, keepdims=True)
    ssum_ref[...] += jnp.sum(jnp.where(mask, s, 0.0), axis=0, keepdims=True)


def _node_update(nvalid, ne, s, cnt, wm, bm, v1n, v1m, cnrow, v2, b2, v3, b3,
                 lg, lb):
    npad = ne.shape[0]
    nblk = npad // BN
    row = pl.BlockSpec((1, D), lambda i: (0, 0))
    mat = pl.BlockSpec((D, D), lambda i: (0, 0))
    blk = pl.BlockSpec((BN, D), lambda i: (i, 0))
    cblk = pl.BlockSpec((BN, 128), lambda i: (i, 0))
    return pl.pallas_call(
        _node_body,
        grid=(nblk,),
        in_specs=[pl.BlockSpec(memory_space=pltpu.SMEM),
                  blk, blk, cblk, mat, row, mat, mat, row, mat, row, mat, row,
                  row, row],
        out_specs=[blk, row, row],
        out_shape=[
            jax.ShapeDtypeStruct((npad, D), jnp.float32),
            jax.ShapeDtypeStruct((1, D), jnp.float32),
            jax.ShapeDtypeStruct((1, D), jnp.float32),
        ],
    )(nvalid, ne, s, cnt, wm, bm, v1n, v1m, cnrow, v2, b2, v3, b3, lg, lb)


def _global_body(nreal_ref, ereal_ref, g_ref, ns_ref, ss_ref, wng_ref,
                 bng_ref, weg_ref, beg_ref, u1g_ref, u1n_ref, u1e_ref, ub1_ref,
                 u2_ref, ub2_ref, u3_ref, ub3_ref, lg_ref, lb_ref, out_ref):
    gl = g_ref[...]
    na = _f32dot(ns_ref[...], wng_ref[...]) + nreal_ref[0] * bng_ref[...]
    ea = _f32dot(ss_ref[...], weg_ref[...]) + ereal_ref[0] * beg_ref[...]
    h1 = jnp.maximum(
        _f32dot(gl, u1g_ref[...]) + _f32dot(na, u1n_ref[...]) +
        _f32dot(ea, u1e_ref[...]) + ub1_ref[...], 0.0)
    h2 = jnp.maximum(_f32dot(h1, u2_ref[...]) + ub2_ref[...], 0.0)
    u = _f32dot(h2, u3_ref[...]) + ub3_ref[...]
    out_ref[...] = _layer_norm(gl + u, lg_ref[...], lb_ref[...])


def _global_update(nreal, ereal, g_emb, nsum, ssum, wng, bng, weg, beg, u1g,
                   u1n, u1e, ub1, u2, ub2, u3, ub3, lg, lb):
    row = pl.BlockSpec((1, D), lambda: (0, 0))
    mat = pl.BlockSpec((D, D), lambda: (0, 0))
    return pl.pallas_call(
        _global_body,
        grid=(),
        in_specs=[pl.BlockSpec(memory_space=pltpu.SMEM),
                  pl.BlockSpec(memory_space=pltpu.SMEM),
                  row, row, row, mat, row, mat, row, mat, mat, mat, row, mat,
                  row, mat, row, row, row],
        out_specs=row,
        out_shape=jax.ShapeDtypeStruct((1, D), jnp.float32),
    )(nreal, ereal, g_emb, nsum, ssum, wng, bng, weg, beg, u1g, u1n, u1e, ub1,
      u2, ub2, u3, ub3, lg, lb)


def _sc_gather2(a, b, src2, dst2):
    nchunks = src2.shape[0]
    epad = nchunks * CH
    nch = nchunks // NW

    @functools.partial(
        pl.kernel,
        mesh=_sc_mesh(),
        out_type=[jax.ShapeDtypeStruct((epad, D), jnp.float32),
                  jax.ShapeDtypeStruct((epad, D), jnp.float32)],
        scratch_types=[pltpu.VMEM((nch, CH), jnp.int32),
                       pltpu.VMEM((nch, CH), jnp.int32),
                       pltpu.VMEM((CH, D), jnp.float32),
                       pltpu.VMEM((CH, D), jnp.float32),
                       pltpu.SemaphoreType.DMA,
                       pltpu.SemaphoreType.DMA],
    )
    def k(a_hbm, b_hbm, s_hbm, d_hbm, ga_hbm, gb_hbm, sidx, didx, bufa, bufb,
          sema, semb):
        w = lax.axis_index("s") * 2 + lax.axis_index("c")
        c0 = w * nch
        pltpu.sync_copy(s_hbm.at[pl.ds(c0, nch)], sidx)
        pltpu.sync_copy(d_hbm.at[pl.ds(c0, nch)], didx)

        def body(j, carry):
            ca = pltpu.async_copy(a_hbm.at[sidx.at[j]], bufa, sema)
            cb = pltpu.async_copy(b_hbm.at[didx.at[j]], bufb, semb)
            ca.wait()
            cb.wait()
            base = (c0 + j) * CH
            pltpu.sync_copy(bufa, ga_hbm.at[pl.ds(base, CH)])
            pltpu.sync_copy(bufb, gb_hbm.at[pl.ds(base, CH)])
            return carry

        lax.fori_loop(0, nch, body, 0)

    return k(a, b, src2, dst2)


def _sc_scatter(vals, dst2, zrows):
    nchunks = dst2.shape[0]
    npad = zrows.shape[0]
    nch = nchunks // NT
    rows_t = npad // NT

    @functools.partial(
        pl.kernel,
        mesh=_sc_mesh(),
        out_type=jax.ShapeDtypeStruct((npad, D), jnp.float32),
        scratch_types=[pltpu.VMEM((nch, CH), jnp.int32),
                       pltpu.VMEM((CH, 128), jnp.float32),
                       pltpu.VMEM_SHARED((npad, 128), jnp.float32)],
    )
    def k(v_hbm, d_hbm, z_hbm, s_hbm, didx, vbuf, acc):
        c = lax.axis_index("c")
        t = lax.axis_index("s")
        pltpu.sync_copy(z_hbm.at[pl.ds(t * rows_t, rows_t)],
                        acc.at[pl.ds(t * rows_t, rows_t)])
        pltpu.sync_copy(d_hbm.at[pl.ds(t * nch, nch)], didx)
        plsc.subcore_barrier()

        def body(j, carry):
            ch = t * nch + j
            pltpu.sync_copy(
                v_hbm.at[pl.ds(ch * CH, CH), pl.ds(c * 128, 128)], vbuf)
            pltpu.sync_copy(vbuf, acc.at[didx.at[j]], add=True)
            return carry

        lax.fori_loop(0, nch, body, 0)
        plsc.subcore_barrier()
        pltpu.sync_copy(acc.at[pl.ds(t * rows_t, rows_t)],
                        s_hbm.at[pl.ds(t * rows_t, rows_t),
                                 pl.ds(c * 128, 128)])

    return k(vals, dst2, zrows)


def _sc_counts(dst2, ones_rows, zrows):
    nchunks = dst2.shape[0]
    npad = zrows.shape[0]
    nch = nchunks // NT
    rows_t = npad // NT

    @functools.partial(
        pl.kernel,
        mesh=_sc_mesh(),
        out_type=jax.ShapeDtypeStruct((npad, 128), jnp.float32),
        scratch_types=[pltpu.VMEM((nch, CH), jnp.int32),
                       pltpu.VMEM((CH, 128), jnp.float32),
                       pltpu.VMEM_SHARED((npad, 128), jnp.float32)],
    )
    def k(d_hbm, o_hbm, z_hbm, cnt_hbm, didx, vbuf, acc):
        c = lax.axis_index("c")
        t = lax.axis_index("s")

        @pl.when(c == 0)
        def _():
            pltpu.sync_copy(z_hbm.at[pl.ds(t * rows_t, rows_t)],
                            acc.at[pl.ds(t * rows_t, rows_t)])
            pltpu.sync_copy(d_hbm.at[pl.ds(t * nch, nch)], didx)
            pltpu.sync_copy(o_hbm, vbuf)
            plsc.subcore_barrier()

            def body(j, carry):
                pltpu.sync_copy(vbuf, acc.at[didx.at[j]], add=True)
                return carry

            lax.fori_loop(0, nch, body, 0)
            plsc.subcore_barrier()
            pltpu.sync_copy(acc.at[pl.ds(t * rows_t, rows_t)],
                            cnt_hbm.at[pl.ds(t * rows_t, rows_t)])

    return k(dst2, ones_rows, zrows)


def _sc_init_node(temb, kemb, x0_2, x1_2):
    nchunks = x0_2.shape[0]
    npad = nchunks * CH
    nloop = (nchunks + NW - 1) // NW

    @functools.partial(
        pl.kernel,
        mesh=_sc_mesh(),
        out_type=jax.ShapeDtypeStruct((npad, D), jnp.float32),
        scratch_types=[pltpu.VMEM((1, CH), jnp.int32),
                       pltpu.VMEM((1, CH), jnp.int32),
                       pltpu.VMEM((CH, 128), jnp.float32),
                       pltpu.VMEM((CH, 128), jnp.float32),
                       pltpu.SemaphoreType.DMA,
                       pltpu.SemaphoreType.DMA],
    )
    def k(t_hbm, k_hbm, x0_hbm, x1_hbm, out_hbm, i0, i1, buft, bufk, sema,
          semb):
        w = lax.axis_index("s") * 2 + lax.axis_index("c")
        for j in range(nloop):
            ch = w + NW * j

            @pl.when(ch < nchunks)
            def _():
                pltpu.sync_copy(x0_hbm.at[pl.ds(ch, 1)], i0)
                pltpu.sync_copy(x1_hbm.at[pl.ds(ch, 1)], i1)
                ca = pltpu.async_copy(t_hbm.at[i0.at[0]], buft, sema)
                cb = pltpu.async_copy(k_hbm.at[i1.at[0]], bufk, semb)
                ca.wait()
                cb.wait()
                pltpu.sync_copy(
                    buft, out_hbm.at[pl.ds(ch * CH, CH), pl.ds(0, 128)])
                pltpu.sync_copy(
                    bufk, out_hbm.at[pl.ds(ch * CH, CH), pl.ds(128, 128)])

    return k(temb, kemb, x0_2, x1_2)


def _sc_init_edge(etab, ea_2):
    nchunks = ea_2.shape[0]
    epad = nchunks * CH
    nch = nchunks // NW

    @functools.partial(
        pl.kernel,
        mesh=_sc_mesh(),
        out_type=jax.ShapeDtypeStruct((epad, D), jnp.float32),
        scratch_types=[pltpu.VMEM((nch, CH), jnp.int32),
                       pltpu.VMEM((CH, D), jnp.float32),
                       pltpu.SemaphoreType.DMA],
    )
    def k(tab_hbm, e_hbm, out_hbm, eidx, buf, sem):
        w = lax.axis_index("s") * 2 + lax.axis_index("c")
        c0 = w * nch
        pltpu.sync_copy(e_hbm.at[pl.ds(c0, nch)], eidx)

        def body(j, carry):
            pltpu.async_copy(tab_hbm.at[eidx.at[j]], buf, sem).wait()
            pltpu.sync_copy(buf, out_hbm.at[pl.ds((c0 + j) * CH, CH)])
            return carry

        lax.fori_loop(0, nch, body, 0)

    return k(etab, ea_2)


def kernel(x, edge_index, edge_attr, params):
    n = x.shape[0]
    e = edge_index.shape[1]
    egran = CH * NW
    ngran = BN
    npad = ((n + ngran - 1) // ngran) * ngran
    epad = ((e + egran - 1) // egran) * egran

    src2 = jnp.pad(edge_index[0], (0, epad - e)).reshape(epad // CH, CH)
    dst2 = jnp.pad(edge_index[1], (0, epad - e),
                   constant_values=n).reshape(epad // CH, CH)
    ea2 = jnp.pad(edge_attr[:, 0], (0, epad - e)).reshape(epad // CH, CH)
    x0_2 = jnp.pad(x[:, 0], (0, npad - n)).reshape(npad // CH, CH)
    x1_2 = jnp.pad(x[:, 1], (0, npad - n)).reshape(npad // CH, CH)
    zrows = jnp.zeros((npad, 128), jnp.float32)
    ones_rows = jnp.ones((CH, 128), jnp.float32)

    node_emb = _sc_init_node(params["node_type_emb"], params["node_token_emb"],
                             x0_2, x1_2)
    edge_emb = _sc_init_edge(params["edge_type_emb"], ea2)
    gi = params["global_init"]
    g_emb = (jnp.ones((1, 1), jnp.float32) @ gi["w"] + gi["b"])

    counts = _sc_counts(dst2, ones_rows, zrows)

    nreal = jnp.array([n], jnp.float32)
    ereal = jnp.array([e], jnp.float32)
    nvalid = jnp.array([n], jnp.int32)

    for lp in params["layers"]:
        w1 = lp["edge_mlp"][0]["w"]
        w1s, w1d, w1e, w1g = w1[:D], w1[D:2 * D], w1[2 * D:3 * D], w1[3 * D:]
        b1 = lp["edge_mlp"][0]["b"][None, :]
        v1 = lp["node_mlp"][0]["w"]
        v1n, v1m, v1g = v1[:D], v1[D:2 * D], v1[2 * D:]
        bn1 = lp["node_mlp"][0]["b"][None, :]
        u1 = lp["global_mlp"][0]["w"]
        u1g, u1n, u1e = u1[:D], u1[D:2 * D], u1[2 * D:]

        a, b, crow, cnrow = _node_proj(node_emb, g_emb, w1s, w1d, w1g, b1,
                                       v1g, bn1)
        ga, gb = _sc_gather2(a, b, src2, dst2)
        edge_emb = _edge_update(
            ga, gb, edge_emb, w1e, crow,
            lp["edge_mlp"][1]["w"], lp["edge_mlp"][1]["b"][None, :],
            lp["edge_mlp"][2]["w"], lp["edge_mlp"][2]["b"][None, :],
            lp["ln_edge"]["g"][None, :], lp["ln_edge"]["b"][None, :])
        s = _sc_scatter(edge_emb, dst2, zrows)
        node_emb, nsum, ssum = _node_update(
            nvalid, node_emb, s, counts,
            lp["edge_to_message"]["w"], lp["edge_to_message"]["b"][None, :],
            v1n, v1m, cnrow,
            lp["node_mlp"][1]["w"], lp["node_mlp"][1]["b"][None, :],
            lp["node_mlp"][2]["w"], lp["node_mlp"][2]["b"][None, :],
            lp["ln_node"]["g"][None, :], lp["ln_node"]["b"][None, :])
        g_emb = _global_update(
            nreal, ereal, g_emb, nsum, ssum,
            lp["node_to_global"]["w"], lp["node_to_global"]["b"][None, :],
            lp["edge_to_global"]["w"], lp["edge_to_global"]["b"][None, :],
            u1g, u1n, u1e, lp["global_mlp"][0]["b"][None, :],
            lp["global_mlp"][1]["w"], lp["global_mlp"][1]["b"][None, :],
            lp["global_mlp"][2]["w"], lp["global_mlp"][2]["b"][None, :],
            lp["ln_global"]["g"][None, :], lp["ln_global"]["b"][None, :])

    return node_emb[:n], edge_emb[:e], g_emb


# speedup vs baseline: 2.4889x; 1.2768x over previous
import functools

import jax
import jax.numpy as jnp
from jax import lax
from jax.experimental import pallas as pl
from jax.experimental.pallas import tpu as pltpu
from jax.experimental.pallas import tpu_sc as plsc

D = 256
BN = 2048
BE = 2048
CH = 128
NW = 32
NT = 16


def _sc_mesh():
    return plsc.VectorSubcoreMesh(core_axis_name="c", subcore_axis_name="s")


def _f32dot(a, b):
    return jnp.dot(a, b, preferred_element_type=jnp.float32)


def _layer_norm(r, g, b):
    m = jnp.mean(r, axis=-1, keepdims=True)
    d = r - m
    v = jnp.mean(d * d, axis=-1, keepdims=True)
    return d * lax.rsqrt(v + 1e-5) * g + b


def _proj_body(ne_ref, w1s_ref, w1d_ref, g_ref, w1g_ref, b1_ref, v1g_ref,
               bn1_ref, a_ref, b_ref, crow_ref, cnrow_ref):
    ne = ne_ref[...]
    a_ref[...] = _f32dot(ne, w1s_ref[...])
    b_ref[...] = _f32dot(ne, w1d_ref[...])

    @pl.when(pl.program_id(0) == 0)
    def _():
        gl = g_ref[...]
        crow_ref[...] = _f32dot(gl, w1g_ref[...]) + b1_ref[...]
        cnrow_ref[...] = _f32dot(gl, v1g_ref[...]) + bn1_ref[...]


def _node_proj(node_emb, g_emb, w1s, w1d, w1g, b1, v1g, bn1):
    npad = node_emb.shape[0]
    nblk = npad // BN
    row = pl.BlockSpec((1, D), lambda i: (0, 0))
    mat = pl.BlockSpec((D, D), lambda i: (0, 0))
    blk = pl.BlockSpec((BN, D), lambda i: (i, 0))
    return pl.pallas_call(
        _proj_body,
        grid=(nblk,),
        in_specs=[blk, mat, mat, row, mat, row, mat, row],
        out_specs=[blk, blk, row, row],
        out_shape=[
            jax.ShapeDtypeStruct((npad, D), jnp.float32),
            jax.ShapeDtypeStruct((npad, D), jnp.float32),
            jax.ShapeDtypeStruct((1, D), jnp.float32),
            jax.ShapeDtypeStruct((1, D), jnp.float32),
        ],
    )(node_emb, w1s, w1d, g_emb, w1g, b1, v1g, bn1)


def _edge_mlp_ln(ee, ga, gb, w1e_ref, crow_ref, w2_ref, b2_ref, w3_ref,
                 b3_ref, lg_ref, lb_ref):
    h1 = jnp.maximum(ga + gb + _f32dot(ee, w1e_ref[...]) + crow_ref[...], 0.0)
    h2 = jnp.maximum(_f32dot(h1, w2_ref[...]) + b2_ref[...], 0.0)
    u = _f32dot(h2, w3_ref[...]) + b3_ref[...]
    return _layer_norm(ee + u, lg_ref[...], lb_ref[...])


def _edge_body(ga_ref, gb_ref, ee_ref, w1e_ref, crow_ref, w2_ref, b2_ref,
               w3_ref, b3_ref, lg_ref, lb_ref, out_ref):
    out_ref[...] = _edge_mlp_ln(ee_ref[...], ga_ref[...], gb_ref[...],
                                w1e_ref, crow_ref, w2_ref, b2_ref, w3_ref,
                                b3_ref, lg_ref, lb_ref)


def _edge_body_l1(ga_ref, gb_ref, ea_ref, etab_ref, w1e_ref, crow_ref,
                  w2_ref, b2_ref, w3_ref, b3_ref, lg_ref, lb_ref, out_ref):
    oh = (ea_ref[...] == lax.broadcasted_iota(jnp.int32, (BE, 128), 1)
          ).astype(jnp.float32)
    ee = _f32dot(oh, etab_ref[...])
    out_ref[...] = _edge_mlp_ln(ee, ga_ref[...], gb_ref[...], w1e_ref,
                                crow_ref, w2_ref, b2_ref, w3_ref, b3_ref,
                                lg_ref, lb_ref)


def _edge_update(ga, gb, ee_or_ea, etab, w1e, crow, w2, b2, w3, b3, lg, lb):
    epad = ga.shape[0]
    nblk = epad // BE
    row = pl.BlockSpec((1, D), lambda i: (0, 0))
    mat = pl.BlockSpec((D, D), lambda i: (0, 0))
    blk = pl.BlockSpec((BE, D), lambda i: (i, 0))
    out = jax.ShapeDtypeStruct((epad, D), jnp.float32)
    if etab is None:
        return pl.pallas_call(
            _edge_body,
            grid=(nblk,),
            in_specs=[blk, blk, blk, mat, row, mat, row, mat, row, row, row],
            out_specs=blk,
            out_shape=out,
        )(ga, gb, ee_or_ea, w1e, crow, w2, b2, w3, b3, lg, lb)
    iblk = pl.BlockSpec((BE, 1), lambda i: (i, 0))
    etb = pl.BlockSpec((128, D), lambda i: (0, 0))
    return pl.pallas_call(
        _edge_body_l1,
        grid=(nblk,),
        in_specs=[blk, blk, iblk, etb, mat, row, mat, row, mat, row, row,
                  row],
        out_specs=blk,
        out_shape=out,
    )(ga, gb, ee_or_ea, etab, w1e, crow, w2, b2, w3, b3, lg, lb)


def _init_node_body(x0_ref, x1_ref, tpad_ref, kpad_ref, out_ref):
    oh_t = (x0_ref[...] == lax.broadcasted_iota(jnp.int32, (BN, 128), 1)
            ).astype(jnp.float32)
    out_ref[:, :128] = _f32dot(oh_t, tpad_ref[...])
    oh_k = (x1_ref[...] == lax.broadcasted_iota(jnp.int32, (BN, 1024), 1)
            ).astype(jnp.float32)
    out_ref[:, 128:] = _f32dot(oh_k, kpad_ref[...])


def _init_node_tc(x0c, x1c, tpad, kpad):
    npad = x0c.shape[0]
    nblk = npad // BN
    iblk = pl.BlockSpec((BN, 1), lambda i: (i, 0))
    return pl.pallas_call(
        _init_node_body,
        grid=(nblk,),
        in_specs=[iblk, iblk,
                  pl.BlockSpec((128, 128), lambda i: (0, 0)),
                  pl.BlockSpec((1024, 128), lambda i: (0, 0))],
        out_specs=pl.BlockSpec((BN, D), lambda i: (i, 0)),
        out_shape=jax.ShapeDtypeStruct((npad, D), jnp.float32),
    )(x0c, x1c, tpad, kpad)


def _node_body(nvalid_ref, ne_ref, s_ref, cnt0_ref, cnt1_ref, wm_ref, bm_ref,
               v1n_ref, v1m_ref, cnrow_ref, v2_ref, b2_ref, v3_ref, b3_ref,
               lg_ref, lb_ref, out_ref, nsum_ref, ssum_ref):
    ne = ne_ref[...]
    s = s_ref[...]
    cnt = cnt0_ref[...][:, 0:1] + cnt1_ref[...][:, 0:1]
    nm = _f32dot(s, wm_ref[...]) + cnt * bm_ref[...]
    h1 = jnp.maximum(
        _f32dot(ne, v1n_ref[...]) + _f32dot(nm, v1m_ref[...]) + cnrow_ref[...],
        0.0)
    h2 = jnp.maximum(_f32dot(h1, v2_ref[...]) + b2_ref[...], 0.0)
    u = _f32dot(h2, v3_ref[...]) + b3_ref[...]
    out = _layer_norm(ne + u, lg_ref[...], lb_ref[...])
    out_ref[...] = out

    i = pl.program_id(0)
    rows = i * BN + lax.broadcasted_iota(jnp.int32, (BN, 1), 0)
    mask = rows < nvalid_ref[0]

    @pl.when(i == 0)
    def _():
        nsum_ref[...] = jnp.zeros_like(nsum_ref)
        ssum_ref[...] = jnp.zeros_like(ssum_ref)

    nsum_ref[...] += jnp.sum(jnp.where(mask, out, 0.0), axis=0, keepdims=True)
    ssum_ref[...] += jnp.sum(jnp.where(mask, s, 0.0), axis=0, keepdims=True)


def _node_update(nvalid, ne, s, cnt0, cnt1, wm, bm, v1n, v1m, cnrow, v2, b2,
                 v3, b3, lg, lb):
    npad = ne.shape[0]
    nblk = npad // BN
    row = pl.BlockSpec((1, D), lambda i: (0, 0))
    mat = pl.BlockSpec((D, D), lambda i: (0, 0))
    blk = pl.BlockSpec((BN, D), lambda i: (i, 0))
    cblk = pl.BlockSpec((BN, 16), lambda i: (i, 0))
    return pl.pallas_call(
        _node_body,
        grid=(nblk,),
        in_specs=[pl.BlockSpec(memory_space=pltpu.SMEM),
                  blk, blk, cblk, cblk, mat, row, mat, mat, row, mat, row,
                  mat, row, row, row],
        out_specs=[blk, row, row],
        out_shape=[
            jax.ShapeDtypeStruct((npad, D), jnp.float32),
            jax.ShapeDtypeStruct((1, D), jnp.float32),
            jax.ShapeDtypeStruct((1, D), jnp.float32),
        ],
    )(nvalid, ne, s, cnt0, cnt1, wm, bm, v1n, v1m, cnrow, v2, b2, v3, b3,
      lg, lb)


def _global_body(nreal_ref, ereal_ref, g_ref, ns_ref, ss_ref, wng_ref,
                 bng_ref, weg_ref, beg_ref, u1g_ref, u1n_ref, u1e_ref, ub1_ref,
                 u2_ref, ub2_ref, u3_ref, ub3_ref, lg_ref, lb_ref, out_ref):
    gl = g_ref[...]
    na = _f32dot(ns_ref[...], wng_ref[...]) + nreal_ref[0] * bng_ref[...]
    ea = _f32dot(ss_ref[...], weg_ref[...]) + ereal_ref[0] * beg_ref[...]
    h1 = jnp.maximum(
        _f32dot(gl, u1g_ref[...]) + _f32dot(na, u1n_ref[...]) +
        _f32dot(ea, u1e_ref[...]) + ub1_ref[...], 0.0)
    h2 = jnp.maximum(_f32dot(h1, u2_ref[...]) + ub2_ref[...], 0.0)
    u = _f32dot(h2, u3_ref[...]) + ub3_ref[...]
    out_ref[...] = _layer_norm(gl + u, lg_ref[...], lb_ref[...])


def _global_update(nreal, ereal, g_emb, nsum, ssum, wng, bng, weg, beg, u1g,
                   u1n, u1e, ub1, u2, ub2, u3, ub3, lg, lb):
    row = pl.BlockSpec((1, D), lambda: (0, 0))
    mat = pl.BlockSpec((D, D), lambda: (0, 0))
    return pl.pallas_call(
        _global_body,
        grid=(),
        in_specs=[pl.BlockSpec(memory_space=pltpu.SMEM),
                  pl.BlockSpec(memory_space=pltpu.SMEM),
                  row, row, row, mat, row, mat, row, mat, mat, mat, row, mat,
                  row, mat, row, row, row],
        out_specs=row,
        out_shape=jax.ShapeDtypeStruct((1, D), jnp.float32),
    )(nreal, ereal, g_emb, nsum, ssum, wng, bng, weg, beg, u1g, u1n, u1e, ub1,
      u2, ub2, u3, ub3, lg, lb)


def _sc_gather2(a, b, src2, dst2):
    nchunks = src2.shape[0]
    epad = nchunks * CH
    nch = nchunks // NW

    @functools.partial(
        pl.kernel,
        mesh=_sc_mesh(),
        out_type=[jax.ShapeDtypeStruct((epad, D), jnp.float32),
                  jax.ShapeDtypeStruct((epad, D), jnp.float32)],
        scratch_types=[pltpu.VMEM((nch, CH), jnp.int32),
                       pltpu.VMEM((nch, CH), jnp.int32),
                       pltpu.VMEM((CH, D), jnp.float32),
                       pltpu.VMEM((CH, D), jnp.float32),
                       pltpu.SemaphoreType.DMA,
                       pltpu.SemaphoreType.DMA],
    )
    def k(a_hbm, b_hbm, s_hbm, d_hbm, ga_hbm, gb_hbm, sidx, didx, bufa, bufb,
          sema, semb):
        w = lax.axis_index("s") * 2 + lax.axis_index("c")
        c0 = w * nch
        pltpu.sync_copy(s_hbm.at[pl.ds(c0, nch)], sidx)
        pltpu.sync_copy(d_hbm.at[pl.ds(c0, nch)], didx)

        def body(j, carry):
            ca = pltpu.async_copy(a_hbm.at[sidx.at[j]], bufa, sema)
            cb = pltpu.async_copy(b_hbm.at[didx.at[j]], bufb, semb)
            ca.wait()
            cb.wait()
            base = (c0 + j) * CH
            pltpu.sync_copy(bufa, ga_hbm.at[pl.ds(base, CH)])
            pltpu.sync_copy(bufb, gb_hbm.at[pl.ds(base, CH)])
            return carry

        lax.fori_loop(0, nch, body, 0)

    return k(a, b, src2, dst2)


def _sc_scatter(vals, dst2, zrows):
    nchunks = dst2.shape[0]
    npad = zrows.shape[0]
    nch = nchunks // NT
    rows_t = npad // NT

    @functools.partial(
        pl.kernel,
        mesh=_sc_mesh(),
        out_type=jax.ShapeDtypeStruct((npad, D), jnp.float32),
        scratch_types=[pltpu.VMEM((nch, CH), jnp.int32),
                       pltpu.VMEM((CH, 128), jnp.float32),
                       pltpu.VMEM_SHARED((npad, 128), jnp.float32)],
    )
    def k(v_hbm, d_hbm, z_hbm, s_hbm, didx, vbuf, acc):
        c = lax.axis_index("c")
        t = lax.axis_index("s")
        pltpu.sync_copy(z_hbm.at[pl.ds(t * rows_t, rows_t)],
                        acc.at[pl.ds(t * rows_t, rows_t)])
        pltpu.sync_copy(d_hbm.at[pl.ds(t * nch, nch)], didx)
        plsc.subcore_barrier()

        def body(j, carry):
            ch = t * nch + j
            pltpu.sync_copy(
                v_hbm.at[pl.ds(ch * CH, CH), pl.ds(c * 128, 128)], vbuf)
            pltpu.sync_copy(vbuf, acc.at[didx.at[j]], add=True)
            return carry

        lax.fori_loop(0, nch, body, 0)
        plsc.subcore_barrier()
        pltpu.sync_copy(acc.at[pl.ds(t * rows_t, rows_t)],
                        s_hbm.at[pl.ds(t * rows_t, rows_t),
                                 pl.ds(c * 128, 128)])

    return k(vals, dst2, zrows)


def _sc_counts(dst2, ones_rows, zrows16):
    nchunks = dst2.shape[0]
    npad = zrows16.shape[0]
    ncc = nchunks // 2
    nch = ncc // NT
    rows_t = npad // NT

    @functools.partial(
        pl.kernel,
        mesh=_sc_mesh(),
        out_type=[jax.ShapeDtypeStruct((npad, 16), jnp.float32),
                  jax.ShapeDtypeStruct((npad, 16), jnp.float32)],
        scratch_types=[pltpu.VMEM((nch, CH), jnp.int32),
                       pltpu.VMEM((CH, 16), jnp.float32),
                       pltpu.VMEM_SHARED((npad, 16), jnp.float32)],
    )
    def k(d_hbm, o_hbm, z_hbm, cnt0_hbm, cnt1_hbm, didx, vbuf, acc):
        c = lax.axis_index("c")
        t = lax.axis_index("s")
        pltpu.sync_copy(z_hbm.at[pl.ds(t * rows_t, rows_t)],
                        acc.at[pl.ds(t * rows_t, rows_t)])
        pltpu.sync_copy(d_hbm.at[pl.ds(c * ncc + t * nch, nch)], didx)
        pltpu.sync_copy(o_hbm, vbuf)
        plsc.subcore_barrier()

        def body(j, carry):
            pltpu.sync_copy(vbuf, acc.at[didx.at[j]], add=True)
            return carry

        lax.fori_loop(0, nch, body, 0)
        plsc.subcore_barrier()

        @pl.when(c == 0)
        def _():
            pltpu.sync_copy(acc.at[pl.ds(t * rows_t, rows_t)],
                            cnt0_hbm.at[pl.ds(t * rows_t, rows_t)])

        @pl.when(c == 1)
        def _():
            pltpu.sync_copy(acc.at[pl.ds(t * rows_t, rows_t)],
                            cnt1_hbm.at[pl.ds(t * rows_t, rows_t)])

    return k(dst2, ones_rows, zrows16)


def kernel(x, edge_index, edge_attr, params):
    n = x.shape[0]
    e = edge_index.shape[1]
    egran = CH * NW
    ngran = BN
    npad = ((n + ngran - 1) // ngran) * ngran
    epad = ((e + egran - 1) // egran) * egran

    src2 = jnp.pad(edge_index[0], (0, epad - e)).reshape(epad // CH, CH)
    dst2 = jnp.pad(edge_index[1], (0, epad - e),
                   constant_values=n).reshape(epad // CH, CH)
    ea_c = jnp.pad(edge_attr[:, 0], (0, epad - e)).reshape(epad, 1)
    x0_c = jnp.pad(x[:, 0], (0, npad - n)).reshape(npad, 1)
    x1_c = jnp.pad(x[:, 1], (0, npad - n)).reshape(npad, 1)
    zrows = jnp.zeros((npad, 128), jnp.float32)
    zrows16 = jnp.zeros((npad, 16), jnp.float32)
    ones16 = jnp.ones((CH, 16), jnp.float32)
    tpad = jnp.zeros((128, 128), jnp.float32).at[:10].set(
        params["node_type_emb"])
    kpad = jnp.zeros((1024, 128), jnp.float32).at[:1000].set(
        params["node_token_emb"])
    etab = jnp.zeros((128, D), jnp.float32).at[:10].set(
        params["edge_type_emb"])

    node_emb = _init_node_tc(x0_c, x1_c, tpad, kpad)
    gi = params["global_init"]
    g_emb = (jnp.ones((1, 1), jnp.float32) @ gi["w"] + gi["b"])

    cnt0, cnt1 = _sc_counts(dst2, ones16, zrows16)

    nreal = jnp.array([n], jnp.float32)
    ereal = jnp.array([e], jnp.float32)
    nvalid = jnp.array([n], jnp.int32)
    edge_emb = None

    for lp in params["layers"]:
        w1 = lp["edge_mlp"][0]["w"]
        w1s, w1d, w1e, w1g = w1[:D], w1[D:2 * D], w1[2 * D:3 * D], w1[3 * D:]
        b1 = lp["edge_mlp"][0]["b"][None, :]
        v1 = lp["node_mlp"][0]["w"]
        v1n, v1m, v1g = v1[:D], v1[D:2 * D], v1[2 * D:]
        bn1 = lp["node_mlp"][0]["b"][None, :]
        u1 = lp["global_mlp"][0]["w"]
        u1g, u1n, u1e = u1[:D], u1[D:2 * D], u1[2 * D:]

        a, b, crow, cnrow = _node_proj(node_emb, g_emb, w1s, w1d, w1g, b1,
                                       v1g, bn1)
        ga, gb = _sc_gather2(a, b, src2, dst2)
        edge_emb = _edge_update(
            ga, gb, ea_c if edge_emb is None else edge_emb,
            etab if edge_emb is None else None, w1e, crow,
            lp["edge_mlp"][1]["w"], lp["edge_mlp"][1]["b"][None, :],
            lp["edge_mlp"][2]["w"], lp["edge_mlp"][2]["b"][None, :],
            lp["ln_edge"]["g"][None, :], lp["ln_edge"]["b"][None, :])
        s = _sc_scatter(edge_emb, dst2, zrows)
        node_emb, nsum, ssum = _node_update(
            nvalid, node_emb, s, cnt0, cnt1,
            lp["edge_to_message"]["w"], lp["edge_to_message"]["b"][None, :],
            v1n, v1m, cnrow,
            lp["node_mlp"][1]["w"], lp["node_mlp"][1]["b"][None, :],
            lp["node_mlp"][2]["w"], lp["node_mlp"][2]["b"][None, :],
            lp["ln_node"]["g"][None, :], lp["ln_node"]["b"][None, :])
        g_emb = _global_update(
            nreal, ereal, g_emb, nsum, ssum,
            lp["node_to_global"]["w"], lp["node_to_global"]["b"][None, :],
            lp["edge_to_global"]["w"], lp["edge_to_global"]["b"][None, :],
            u1g, u1n, u1e, lp["global_mlp"][0]["b"][None, :],
            lp["global_mlp"][1]["w"], lp["global_mlp"][1]["b"][None, :],
            lp["global_mlp"][2]["w"], lp["global_mlp"][2]["b"][None, :],
            lp["ln_global"]["g"][None, :], lp["ln_global"]["b"][None, :])

    return node_emb[:n], edge_emb[:e], g_emb
